# baseline (speedup 1.0000x reference)
import functools

import jax
import jax.numpy as jnp
from jax.experimental import pallas as pl
from jax.experimental.pallas import tpu as pltpu
from jax.experimental.pallas import tpu_sc as plsc

B = 1024
D = 768
FD = 768
BANK = 1000
NEG = -1e30
SUB = FD // 128
GWS = 128


def _dot(a, b, dims):
    return jax.lax.dot_general(a, b, (dims, ((), ())),
                               precision=jax.lax.Precision.DEFAULT,
                               preferred_element_type=jnp.float32)


def _avail_stats(mi, t):
    missing_row = (mi == t)
    avail_row = jnp.where(missing_row, 0.0, 1.0)
    ii = jax.lax.broadcasted_iota(jnp.int32, (B, B), 0)
    jj = jax.lax.broadcasted_iota(jnp.int32, (B, B), 1)
    triu = jnp.where(ii <= jj, 1.0, 0.0)
    csum = _dot(avail_row, triu, (((1,), (0,))))
    A = jnp.sum(avail_row, keepdims=True)
    off = jnp.maximum(A - BANK, 0.0)
    Ae = jnp.minimum(A, float(BANK))
    return missing_row, avail_row, csum, off, Ae, ii, jj


def _proj_prep_kernel(t, mi_ref, x_ref, w_ref, b_ref, proj_ref, src_ref):
    mi = mi_ref[...]
    missing_row, avail_row, csum, off, Ae, ii, jj = _avail_stats(mi, t)

    proj = _dot(x_ref[...], w_ref[...], (((1,), (0,)))) + b_ref[...]
    for a in range(SUB):
        proj_ref[a] = proj[:, 128 * a:128 * (a + 1)]

    slot = csum - 1.0 - off
    pidx = ii.astype(jnp.float32)
    P = jnp.where((pidx == slot) & (avail_row > 0.0), 1.0, 0.0)
    iotaf = jax.lax.broadcasted_iota(jnp.int32, (B, 1), 0).astype(jnp.float32)
    srcf = jnp.sum(P * jj.astype(jnp.float32), axis=1, keepdims=True)
    has = jnp.sum(P, axis=1, keepdims=True)
    srcf = srcf + (1.0 - has) * iotaf
    src_ref[...] = jnp.transpose(srcf)


def _sc_gather(table, idx):
    n = idx.shape[1]
    mesh = plsc.VectorSubcoreMesh(core_axis_name="core",
                                  subcore_axis_name="subcore")

    @pl.kernel(out_type=jax.ShapeDtypeStruct((n, 128), jnp.float32),
               mesh=mesh)
    def k(tab_hbm, i_hbm, o_hbm):
        def body(i_vmem, o_vmem):
            pltpu.sync_copy(tab_hbm.at[i_vmem.at[0]], o_vmem)

        pltpu.emit_pipeline(
            body,
            grid=(n // GWS,),
            in_specs=[pl.BlockSpec((1, GWS), index_map=lambda i: (0, i))],
            out_specs=[pl.BlockSpec((GWS, 128), index_map=lambda i: (i, 0))],
            core_axis_name=("core", "subcore"),
            dimension_semantics=(pltpu.PARALLEL,),
        )(i_hbm, o_hbm)

    return k(table, idx)


def _main_kernel(mi_ref, f0_ref, f1_ref, f2_ref, p0_ref, p1_ref, p2_ref,
                 w1_ref, b1_ref, w2_ref, b2_ref, out_ref, acc_ref):
    m = pl.program_id(0)
    t = m + 1
    mi = mi_ref[...]
    missing_row = (mi == t)
    avail_row = jnp.where(missing_row, 0.0, 1.0)
    A = jnp.sum(avail_row, keepdims=True)
    Ae = jnp.minimum(A, float(BANK))
    jj = jax.lax.broadcasted_iota(jnp.int32, (B, B), 1)

    def asm(ref):
        return jnp.concatenate([ref[a] for a in range(SUB)], axis=1)

    proj = jnp.where(m == 0, asm(p0_ref),
                     jnp.where(m == 1, asm(p1_ref), asm(p2_ref)))
    Fb = jnp.where(m == 0, asm(f0_ref),
                   jnp.where(m == 1, asm(f1_ref), asm(f2_ref)))

    nrm2 = jnp.sum(Fb * Fb, axis=1, keepdims=True)
    nrm = jnp.sqrt(nrm2)
    dotm = _dot(Fb, Fb, (((1,), (1,))))
    nprod = nrm * jnp.transpose(nrm)
    sim = dotm / jnp.maximum(nprod, 1e-8)

    jrow = jax.lax.broadcasted_iota(jnp.int32, (1, B), 1).astype(jnp.float32)
    colvalid = (avail_row > 0.0) & (jrow < Ae) & (jrow < float(BANK))
    L = jnp.sum(jnp.where(colvalid, 1.0, 0.0), keepdims=True)
    v = jnp.where(colvalid, sim, NEG)

    tvs, tis = [], []
    for _ in range(3):
        mval = jnp.max(v, axis=1, keepdims=True)
        midx = jnp.min(jnp.where(v == mval, jj, B), axis=1, keepdims=True)
        tvs.append(mval)
        tis.append(midx)
        v = jnp.where(jj == midx, NEG, v)

    e2 = jnp.exp(tvs[1] - tvs[0])
    e3 = jnp.exp(tvs[2] - tvs[0])
    den = 1.0 + e2 + e3
    w1 = 1.0 / den
    w2 = e2 / den
    w3 = e3 / den

    S = (jnp.where(jj == tis[0], w1, 0.0)
         + jnp.where(jj == tis[1], w2, 0.0)
         + jnp.where(jj == tis[2], w3, 0.0))
    knn = _dot(S, Fb, (((1,), (0,))))

    icol = jax.lax.broadcasted_iota(jnp.int32, (B, 1), 0).astype(jnp.float32)
    use = (icol < Ae) & (L > 0.0)
    missing_col = jnp.transpose(missing_row)
    knn = jnp.where(use, knn, 0.0)
    filled = jnp.where(missing_col, knn, proj)

    contrib = _dot(filled, w1_ref[...], (((1,), (0,))))

    @pl.when(m == 0)
    def _():
        acc_ref[...] = contrib

    @pl.when(m > 0)
    def _():
        acc_ref[...] = acc_ref[...] + contrib

    @pl.when(m == 2)
    def _():
        h = jnp.maximum(acc_ref[...] + b1_ref[...], 0.0)
        out_ref[...] = _dot(h, w2_ref[...], (((1,), (0,)))) + b2_ref[...]


def _tc_proj_prep_one(m, mi, X, W, b):
    return pl.pallas_call(
        functools.partial(_proj_prep_kernel, m + 1),
        out_shape=[
            jax.ShapeDtypeStruct((SUB, B, 128), jnp.float32),
            jax.ShapeDtypeStruct((1, B), jnp.float32),
        ],
    )(mi, X, W, b)


def _tc_main(mi, Fbs, projs, W1, b1, W2, b2):
    fullT = pl.BlockSpec((SUB, B, 128), lambda m: (0, 0, 0))
    return pl.pallas_call(
        _main_kernel,
        grid=(3,),
        in_specs=[
            pl.BlockSpec((1, B), lambda m: (0, 0)),
            fullT, fullT, fullT, fullT, fullT, fullT,
            pl.BlockSpec((FD, FD), lambda m: (m, 0)),
            pl.BlockSpec((1, FD), lambda m: (0, 0)),
            pl.BlockSpec((FD, 1), lambda m: (0, 0)),
            pl.BlockSpec((1, 1), lambda m: (0, 0)),
        ],
        out_specs=pl.BlockSpec((B, 1), lambda m: (0, 0)),
        out_shape=jax.ShapeDtypeStruct((B, 1), jnp.float32),
        scratch_shapes=[pltpu.VMEM((B, FD), jnp.float32)],
    )(mi, *Fbs, *projs, W1, b1, W2, b2)


def kernel(language, video, audio, missing_index, W_language, b_language,
           W_video, b_video, W_audio, b_audio, W1, b1, W2, b2):
    Xs = [language, video, audio]
    Ws = [W_language, W_video, W_audio]
    bs = [b_language.reshape(1, FD), b_video.reshape(1, FD),
          b_audio.reshape(1, FD)]
    mi = missing_index.astype(jnp.int32).reshape(1, B)

    suboff = jnp.arange(SUB, dtype=jnp.int32)[:, None] * B
    projs, Fbs = [], []
    for m in range(3):
        projT, srcf = _tc_proj_prep_one(m, mi, Xs[m], Ws[m], bs[m])
        src = srcf.reshape(B).astype(jnp.int32)
        idx = (suboff + src[None, :]).reshape(1, B * SUB)
        Fbs.append(_sc_gather(projT.reshape(B * SUB, 128), idx))
        projs.append(projT)
    Fbs = [f.reshape(SUB, B, 128) for f in Fbs]
    return _tc_main(mi, Fbs, projs, W1, b1.reshape(1, FD), W2,
                    b2.reshape(1, 1))


# speedup vs baseline: 1.1216x; 1.1216x over previous
import jax
import jax.numpy as jnp
from jax.experimental import pallas as pl
from jax.experimental.pallas import tpu as pltpu
from jax.experimental.pallas import tpu_sc as plsc

B = 1024
D = 768
FD = 768
BANK = 1000
NEG = -1e30
SUB = FD // 128
GWS = 128


def _dot(a, b, dims):
    return jax.lax.dot_general(a, b, (dims, ((), ())),
                               precision=jax.lax.Precision.DEFAULT,
                               preferred_element_type=jnp.float32)


def _proj_prep_kernel(mi_ref, x_ref, w_ref, b_ref, proj_ref, src_ref):
    m = pl.program_id(0)
    t = m + 1
    mi = mi_ref[...]
    missing_row = (mi == t)
    avail_row = jnp.where(missing_row, 0.0, 1.0)

    proj = _dot(x_ref[0], w_ref[0], (((1,), (0,)))) + b_ref[0]
    for a in range(SUB):
        proj_ref[0, a] = proj[:, 128 * a:128 * (a + 1)]

    ii = jax.lax.broadcasted_iota(jnp.int32, (B, B), 0)
    jj = jax.lax.broadcasted_iota(jnp.int32, (B, B), 1)
    triu = jnp.where(ii <= jj, 1.0, 0.0)
    csum = _dot(avail_row, triu, (((1,), (0,))))
    A = jnp.sum(avail_row, keepdims=True)
    off = jnp.maximum(A - BANK, 0.0)

    slot = csum - 1.0 - off
    pidx = ii.astype(jnp.float32)
    P = jnp.where((pidx == slot) & (avail_row > 0.0), 1.0, 0.0)

    iotaf = jax.lax.broadcasted_iota(jnp.int32, (B, 1), 0).astype(jnp.float32)
    srcf = jnp.sum(P * jj.astype(jnp.float32), axis=1, keepdims=True)
    has = jnp.sum(P, axis=1, keepdims=True)
    srcf = srcf + (1.0 - has) * iotaf
    src_ref[0] = jnp.transpose(srcf)


def _sc_gather(table, idx):
    n = idx.shape[1]
    mesh = plsc.VectorSubcoreMesh(core_axis_name="core",
                                  subcore_axis_name="subcore")

    @pl.kernel(out_type=jax.ShapeDtypeStruct((n, 128), jnp.float32),
               mesh=mesh)
    def k(tab_hbm, i_hbm, o_hbm):
        def body(i_vmem, o_vmem):
            pltpu.sync_copy(tab_hbm.at[i_vmem.at[0]], o_vmem)

        pltpu.emit_pipeline(
            body,
            grid=(n // GWS,),
            in_specs=[pl.BlockSpec((1, GWS), index_map=lambda i: (0, i))],
            out_specs=[pl.BlockSpec((GWS, 128), index_map=lambda i: (i, 0))],
            core_axis_name=("core", "subcore"),
            dimension_semantics=(pltpu.PARALLEL,),
        )(i_hbm, o_hbm)

    return k(table, idx)


def _main_kernel(mi_ref, fb_ref, proj_ref, w1_ref, b1_ref, w2_ref, b2_ref,
                 out_ref, acc_ref):
    m = pl.program_id(0)
    t = m + 1
    mi = mi_ref[...]
    missing_row = (mi == t)
    avail_row = jnp.where(missing_row, 0.0, 1.0)
    A = jnp.sum(avail_row, keepdims=True)
    Ae = jnp.minimum(A, float(BANK))
    jj = jax.lax.broadcasted_iota(jnp.int32, (B, B), 1)

    proj = jnp.concatenate([proj_ref[0, a] for a in range(SUB)], axis=1)
    Fb = jnp.concatenate([fb_ref[0, a] for a in range(SUB)], axis=1)

    nrm2 = jnp.sum(Fb * Fb, axis=1, keepdims=True)
    nrm = jnp.sqrt(nrm2)
    dotm = _dot(Fb, Fb, (((1,), (1,))))
    nprod = nrm * jnp.transpose(nrm)
    sim = dotm / jnp.maximum(nprod, 1e-8)

    jrow = jax.lax.broadcasted_iota(jnp.int32, (1, B), 1).astype(jnp.float32)
    colvalid = (avail_row > 0.0) & (jrow < Ae) & (jrow < float(BANK))
    L = jnp.sum(jnp.where(colvalid, 1.0, 0.0), keepdims=True)
    v = jnp.where(colvalid, sim, NEG)

    tvs, tis = [], []
    for _ in range(3):
        mval = jnp.max(v, axis=1, keepdims=True)
        midx = jnp.min(jnp.where(v == mval, jj, B), axis=1, keepdims=True)
        tvs.append(mval)
        tis.append(midx)
        v = jnp.where(jj == midx, NEG, v)

    e2 = jnp.exp(tvs[1] - tvs[0])
    e3 = jnp.exp(tvs[2] - tvs[0])
    den = 1.0 + e2 + e3
    w1 = 1.0 / den
    w2 = e2 / den
    w3 = e3 / den

    S = (jnp.where(jj == tis[0], w1, 0.0)
         + jnp.where(jj == tis[1], w2, 0.0)
         + jnp.where(jj == tis[2], w3, 0.0))
    knn = _dot(S, Fb, (((1,), (0,))))

    icol = jax.lax.broadcasted_iota(jnp.int32, (B, 1), 0).astype(jnp.float32)
    use = (icol < Ae) & (L > 0.0)
    missing_col = jnp.transpose(missing_row)
    knn = jnp.where(use, knn, 0.0)
    filled = jnp.where(missing_col, knn, proj)

    contrib = _dot(filled, w1_ref[...], (((1,), (0,))))

    @pl.when(m == 0)
    def _():
        acc_ref[...] = contrib

    @pl.when(m > 0)
    def _():
        acc_ref[...] = acc_ref[...] + contrib

    @pl.when(m == 2)
    def _():
        h = jnp.maximum(acc_ref[...] + b1_ref[...], 0.0)
        out_ref[...] = _dot(h, w2_ref[...], (((1,), (0,)))) + b2_ref[...]


def _tc_proj_prep(mi, Xs, Ws, bs):
    return pl.pallas_call(
        _proj_prep_kernel,
        grid=(3,),
        in_specs=[
            pl.BlockSpec((1, B), lambda m: (0, 0)),
            pl.BlockSpec((1, B, D), lambda m: (m, 0, 0)),
            pl.BlockSpec((1, D, FD), lambda m: (m, 0, 0)),
            pl.BlockSpec((1, 1, FD), lambda m: (m, 0, 0)),
        ],
        out_specs=[
            pl.BlockSpec((1, SUB, B, 128), lambda m: (m, 0, 0, 0)),
            pl.BlockSpec((1, 1, B), lambda m: (m, 0, 0)),
        ],
        out_shape=[
            jax.ShapeDtypeStruct((3, SUB, B, 128), jnp.float32),
            jax.ShapeDtypeStruct((3, 1, B), jnp.float32),
        ],
    )(mi, Xs, Ws, bs)


def _tc_main(mi, FbT, projT, W1, b1, W2, b2):
    return pl.pallas_call(
        _main_kernel,
        grid=(3,),
        in_specs=[
            pl.BlockSpec((1, B), lambda m: (0, 0)),
            pl.BlockSpec((1, SUB, B, 128), lambda m: (m, 0, 0, 0)),
            pl.BlockSpec((1, SUB, B, 128), lambda m: (m, 0, 0, 0)),
            pl.BlockSpec((FD, FD), lambda m: (m, 0)),
            pl.BlockSpec((1, FD), lambda m: (0, 0)),
            pl.BlockSpec((FD, 1), lambda m: (0, 0)),
            pl.BlockSpec((1, 1), lambda m: (0, 0)),
        ],
        out_specs=pl.BlockSpec((B, 1), lambda m: (0, 0)),
        out_shape=jax.ShapeDtypeStruct((B, 1), jnp.float32),
        scratch_shapes=[pltpu.VMEM((B, FD), jnp.float32)],
    )(mi, FbT, projT, W1, b1, W2, b2)


def kernel(language, video, audio, missing_index, W_language, b_language,
           W_video, b_video, W_audio, b_audio, W1, b1, W2, b2):
    Xs = jnp.stack([language, video, audio])
    Ws = jnp.stack([W_language, W_video, W_audio])
    bs = jnp.stack([b_language, b_video, b_audio])[:, None, :]
    mi = missing_index.astype(jnp.int32).reshape(1, B)

    projT, srcf = _tc_proj_prep(mi, Xs, Ws, bs)
    src = srcf.reshape(3, B).astype(jnp.int32)
    idx = (jnp.arange(3, dtype=jnp.int32)[:, None, None] * (SUB * B)
           + jnp.arange(SUB, dtype=jnp.int32)[None, :, None] * B
           + src[:, None, :]).reshape(1, 3 * B * SUB)
    FbT = _sc_gather(projT.reshape(3 * B * SUB, 128), idx)
    return _tc_main(mi, FbT.reshape(3, SUB, B, 128), projT, W1,
                    b1.reshape(1, FD), W2, b2.reshape(1, 1))


# speedup vs baseline: 1.1376x; 1.0142x over previous
import jax
import jax.numpy as jnp
from jax.experimental import pallas as pl
from jax.experimental.pallas import tpu as pltpu
from jax.experimental.pallas import tpu_sc as plsc

B = 1024
D = 768
FD = 768
BANK = 1000
NEG = -1e30
SUB = FD // 128
GWS = 128


def _dot(a, b, dims):
    return jax.lax.dot_general(a, b, (dims, ((), ())),
                               precision=jax.lax.Precision.DEFAULT,
                               preferred_element_type=jnp.float32)


def _proj_prep_kernel(mi_ref, x_ref, w_ref, b_ref, proj_ref, src_ref):
    m = pl.program_id(0)
    t = m + 1
    mi = mi_ref[...]
    missing_row = (mi == t)
    avail_row = jnp.where(missing_row, 0.0, 1.0)

    proj = _dot(x_ref[0], w_ref[0], (((1,), (0,)))) + b_ref[0]
    for a in range(SUB):
        proj_ref[0, a] = proj[:, 128 * a:128 * (a + 1)]

    ii = jax.lax.broadcasted_iota(jnp.int32, (B, B), 0)
    jj = jax.lax.broadcasted_iota(jnp.int32, (B, B), 1)
    triu = jnp.where(ii <= jj, 1.0, 0.0)
    csum = _dot(avail_row, triu, (((1,), (0,))))
    A = jnp.sum(avail_row, keepdims=True)
    off = jnp.maximum(A - BANK, 0.0)

    slot = csum - 1.0 - off
    pidx = ii.astype(jnp.float32)
    P = jnp.where((pidx == slot) & (avail_row > 0.0), 1.0, 0.0)

    iotaf = jax.lax.broadcasted_iota(jnp.int32, (B, 1), 0).astype(jnp.float32)
    srcf = jnp.sum(P * jj.astype(jnp.float32), axis=1, keepdims=True)
    has = jnp.sum(P, axis=1, keepdims=True)
    srcf = srcf + (1.0 - has) * iotaf
    srcT = jnp.transpose(srcf)
    base = (m * (SUB * B)).astype(jnp.float32)
    src_ref[0] = jnp.concatenate(
        [srcT + (base + float(a * B)) for a in range(SUB)],
        axis=1).astype(jnp.int32)


def _sc_gather(table, idx):
    n = idx.shape[1]
    mesh = plsc.VectorSubcoreMesh(core_axis_name="core",
                                  subcore_axis_name="subcore")

    @pl.kernel(out_type=jax.ShapeDtypeStruct((n, 128), jnp.float32),
               mesh=mesh)
    def k(tab_hbm, i_hbm, o_hbm):
        def body(i_vmem, o_vmem):
            pltpu.sync_copy(tab_hbm.at[i_vmem.at[0]], o_vmem)

        pltpu.emit_pipeline(
            body,
            grid=(n // GWS,),
            in_specs=[pl.BlockSpec((1, GWS), index_map=lambda i: (0, i))],
            out_specs=[pl.BlockSpec((GWS, 128), index_map=lambda i: (i, 0))],
            core_axis_name=("core", "subcore"),
            dimension_semantics=(pltpu.PARALLEL,),
        )(i_hbm, o_hbm)

    return k(table, idx)


def _main_kernel(mi_ref, fb_ref, proj_ref, w1_ref, b1_ref, w2_ref, b2_ref,
                 out_ref, acc_ref):
    m = pl.program_id(0)
    t = m + 1
    mi = mi_ref[...]
    missing_row = (mi == t)
    avail_row = jnp.where(missing_row, 0.0, 1.0)
    A = jnp.sum(avail_row, keepdims=True)
    Ae = jnp.minimum(A, float(BANK))
    jj = jax.lax.broadcasted_iota(jnp.int32, (B, B), 1)

    proj = jnp.concatenate([proj_ref[0, a] for a in range(SUB)], axis=1)
    Fb = jnp.concatenate([fb_ref[0, a] for a in range(SUB)], axis=1)

    nrm2 = jnp.sum(Fb * Fb, axis=1, keepdims=True)
    nrm = jnp.sqrt(nrm2)
    dotm = _dot(Fb, Fb, (((1,), (1,))))
    nprod = nrm * jnp.transpose(nrm)
    sim = dotm / jnp.maximum(nprod, 1e-8)

    jrow = jax.lax.broadcasted_iota(jnp.int32, (1, B), 1).astype(jnp.float32)
    colvalid = (avail_row > 0.0) & (jrow < Ae) & (jrow < float(BANK))
    L = jnp.sum(jnp.where(colvalid, 1.0, 0.0), keepdims=True)
    v = jnp.where(colvalid, sim, NEG)

    tvs, tis = [], []
    for _ in range(3):
        mval = jnp.max(v, axis=1, keepdims=True)
        midx = jnp.min(jnp.where(v == mval, jj, B), axis=1, keepdims=True)
        tvs.append(mval)
        tis.append(midx)
        v = jnp.where(jj == midx, NEG, v)

    e2 = jnp.exp(tvs[1] - tvs[0])
    e3 = jnp.exp(tvs[2] - tvs[0])
    den = 1.0 + e2 + e3
    w1 = 1.0 / den
    w2 = e2 / den
    w3 = e3 / den

    S = (jnp.where(jj == tis[0], w1, 0.0)
         + jnp.where(jj == tis[1], w2, 0.0)
         + jnp.where(jj == tis[2], w3, 0.0))
    knn = _dot(S, Fb, (((1,), (0,))))

    icol = jax.lax.broadcasted_iota(jnp.int32, (B, 1), 0).astype(jnp.float32)
    use = (icol < Ae) & (L > 0.0)
    missing_col = jnp.transpose(missing_row)
    knn = jnp.where(use, knn, 0.0)
    filled = jnp.where(missing_col, knn, proj)

    contrib = _dot(filled, w1_ref[...], (((1,), (0,))))

    @pl.when(m == 0)
    def _():
        acc_ref[...] = contrib

    @pl.when(m > 0)
    def _():
        acc_ref[...] = acc_ref[...] + contrib

    @pl.when(m == 2)
    def _():
        h = jnp.maximum(acc_ref[...] + b1_ref[...], 0.0)
        out_ref[...] = _dot(h, w2_ref[...], (((1,), (0,)))) + b2_ref[...]


def _tc_proj_prep(mi, Xs, Ws, bs):
    return pl.pallas_call(
        _proj_prep_kernel,
        grid=(3,),
        in_specs=[
            pl.BlockSpec((1, B), lambda m: (0, 0)),
            pl.BlockSpec((1, B, D), lambda m: (m, 0, 0)),
            pl.BlockSpec((1, D, FD), lambda m: (m, 0, 0)),
            pl.BlockSpec((1, 1, FD), lambda m: (m, 0, 0)),
        ],
        out_specs=[
            pl.BlockSpec((1, SUB, B, 128), lambda m: (m, 0, 0, 0)),
            pl.BlockSpec((1, 1, SUB * B), lambda m: (m, 0, 0)),
        ],
        out_shape=[
            jax.ShapeDtypeStruct((3, SUB, B, 128), jnp.float32),
            jax.ShapeDtypeStruct((3, 1, SUB * B), jnp.int32),
        ],
    )(mi, Xs, Ws, bs)


def _tc_main(mi, FbT, projT, W1, b1, W2, b2):
    return pl.pallas_call(
        _main_kernel,
        grid=(3,),
        in_specs=[
            pl.BlockSpec((1, B), lambda m: (0, 0)),
            pl.BlockSpec((1, SUB, B, 128), lambda m: (m, 0, 0, 0)),
            pl.BlockSpec((1, SUB, B, 128), lambda m: (m, 0, 0, 0)),
            pl.BlockSpec((FD, FD), lambda m: (m, 0)),
            pl.BlockSpec((1, FD), lambda m: (0, 0)),
            pl.BlockSpec((FD, 1), lambda m: (0, 0)),
            pl.BlockSpec((1, 1), lambda m: (0, 0)),
        ],
        out_specs=pl.BlockSpec((B, 1), lambda m: (0, 0)),
        out_shape=jax.ShapeDtypeStruct((B, 1), jnp.float32),
        scratch_shapes=[pltpu.VMEM((B, FD), jnp.float32)],
    )(mi, FbT, projT, W1, b1, W2, b2)


def kernel(language, video, audio, missing_index, W_language, b_language,
           W_video, b_video, W_audio, b_audio, W1, b1, W2, b2):
    Xs = jnp.stack([language, video, audio])
    Ws = jnp.stack([W_language, W_video, W_audio])
    bs = jnp.stack([b_language, b_video, b_audio])[:, None, :]
    mi = missing_index.astype(jnp.int32).reshape(1, B)

    projT, src6 = _tc_proj_prep(mi, Xs, Ws, bs)
    idx = src6.reshape(1, 3 * B * SUB)
    FbT = _sc_gather(projT.reshape(3 * B * SUB, 128), idx)
    return _tc_main(mi, FbT.reshape(3, SUB, B, 128), projT, W1,
                    b1.reshape(1, FD), W2, b2.reshape(1, 1))


# speedup vs baseline: 1.1455x; 1.0070x over previous
import jax
import jax.numpy as jnp
from jax.experimental import pallas as pl
from jax.experimental.pallas import tpu as pltpu
from jax.experimental.pallas import tpu_sc as plsc

B = 1024
D = 768
FD = 768
BANK = 1000
NEG = -1e30
SUB = FD // 128
GWS = 256


def _dot(a, b, dims):
    return jax.lax.dot_general(a, b, (dims, ((), ())),
                               precision=jax.lax.Precision.DEFAULT,
                               preferred_element_type=jnp.float32)


def _proj_prep_kernel(mi_ref, x_ref, w_ref, b_ref, proj_ref, src_ref):
    m = pl.program_id(0)
    t = m + 1
    mi = mi_ref[...]
    missing_row = (mi == t)
    avail_row = jnp.where(missing_row, 0.0, 1.0)

    proj = _dot(x_ref[0], w_ref[0], (((1,), (0,)))) + b_ref[0]
    for a in range(SUB):
        proj_ref[0, a] = proj[:, 128 * a:128 * (a + 1)]

    ii = jax.lax.broadcasted_iota(jnp.int32, (B, B), 0)
    jj = jax.lax.broadcasted_iota(jnp.int32, (B, B), 1)
    triu = jnp.where(ii <= jj, 1.0, 0.0)
    csum = _dot(avail_row, triu, (((1,), (0,))))
    A = jnp.sum(avail_row, keepdims=True)
    off = jnp.maximum(A - BANK, 0.0)

    slot = csum - 1.0 - off
    pidx = ii.astype(jnp.float32)
    P = jnp.where((pidx == slot) & (avail_row > 0.0), 1.0, 0.0)

    iotaf = jax.lax.broadcasted_iota(jnp.int32, (B, 1), 0).astype(jnp.float32)
    srcf = jnp.sum(P * jj.astype(jnp.float32), axis=1, keepdims=True)
    has = jnp.sum(P, axis=1, keepdims=True)
    srcf = srcf + (1.0 - has) * iotaf
    srcT = jnp.transpose(srcf)
    base = (m * (SUB * B)).astype(jnp.float32)
    src_ref[0] = jnp.concatenate(
        [srcT + (base + float(a * B)) for a in range(SUB)],
        axis=1).astype(jnp.int32)


def _sc_gather(table, idx):
    n = idx.shape[1]
    mesh = plsc.VectorSubcoreMesh(core_axis_name="core",
                                  subcore_axis_name="subcore")

    @pl.kernel(out_type=jax.ShapeDtypeStruct((n, 128), jnp.float32),
               mesh=mesh)
    def k(tab_hbm, i_hbm, o_hbm):
        def body(i_vmem, o_vmem):
            pltpu.sync_copy(tab_hbm.at[i_vmem.at[0]], o_vmem)

        pltpu.emit_pipeline(
            body,
            grid=(n // GWS,),
            in_specs=[pl.BlockSpec((1, GWS), index_map=lambda i: (0, i))],
            out_specs=[pl.BlockSpec((GWS, 128), index_map=lambda i: (i, 0))],
            core_axis_name=("core", "subcore"),
            dimension_semantics=(pltpu.PARALLEL,),
        )(i_hbm, o_hbm)

    return k(table, idx)


def _main_kernel(mi_ref, fb_ref, proj_ref, w1_ref, b1_ref, w2_ref, b2_ref,
                 out_ref, acc_ref):
    m = pl.program_id(0)
    t = m + 1
    mi = mi_ref[...]
    missing_row = (mi == t)
    avail_row = jnp.where(missing_row, 0.0, 1.0)
    A = jnp.sum(avail_row, keepdims=True)
    Ae = jnp.minimum(A, float(BANK))
    jj = jax.lax.broadcasted_iota(jnp.int32, (B, B), 1)

    proj = jnp.concatenate([proj_ref[0, a] for a in range(SUB)], axis=1)
    Fb = jnp.concatenate([fb_ref[0, a] for a in range(SUB)], axis=1)

    nrm2 = jnp.sum(Fb * Fb, axis=1, keepdims=True)
    nrm = jnp.sqrt(nrm2)
    dotm = _dot(Fb, Fb, (((1,), (1,))))
    nprod = nrm * jnp.transpose(nrm)
    sim = dotm / jnp.maximum(nprod, 1e-8)

    jrow = jax.lax.broadcasted_iota(jnp.int32, (1, B), 1).astype(jnp.float32)
    colvalid = (avail_row > 0.0) & (jrow < Ae) & (jrow < float(BANK))
    L = jnp.sum(jnp.where(colvalid, 1.0, 0.0), keepdims=True)
    v = jnp.where(colvalid, sim, NEG)

    tvs, tis = [], []
    for _ in range(3):
        mval = jnp.max(v, axis=1, keepdims=True)
        midx = jnp.min(jnp.where(v == mval, jj, B), axis=1, keepdims=True)
        tvs.append(mval)
        tis.append(midx)
        v = jnp.where(jj == midx, NEG, v)

    e2 = jnp.exp(tvs[1] - tvs[0])
    e3 = jnp.exp(tvs[2] - tvs[0])
    den = 1.0 + e2 + e3
    w1 = 1.0 / den
    w2 = e2 / den
    w3 = e3 / den

    S = (jnp.where(jj == tis[0], w1, 0.0)
         + jnp.where(jj == tis[1], w2, 0.0)
         + jnp.where(jj == tis[2], w3, 0.0))
    knn = _dot(S, Fb, (((1,), (0,))))

    icol = jax.lax.broadcasted_iota(jnp.int32, (B, 1), 0).astype(jnp.float32)
    use = (icol < Ae) & (L > 0.0)
    missing_col = jnp.transpose(missing_row)
    knn = jnp.where(use, knn, 0.0)
    filled = jnp.where(missing_col, knn, proj)

    contrib = _dot(filled, w1_ref[...], (((1,), (0,))))

    @pl.when(m == 0)
    def _():
        acc_ref[...] = contrib

    @pl.when(m > 0)
    def _():
        acc_ref[...] = acc_ref[...] + contrib

    @pl.when(m == 2)
    def _():
        h = jnp.maximum(acc_ref[...] + b1_ref[...], 0.0)
        out_ref[...] = _dot(h, w2_ref[...], (((1,), (0,)))) + b2_ref[...]


def _tc_proj_prep(mi, Xs, Ws, bs):
    return pl.pallas_call(
        _proj_prep_kernel,
        grid=(3,),
        in_specs=[
            pl.BlockSpec((1, B), lambda m: (0, 0)),
            pl.BlockSpec((1, B, D), lambda m: (m, 0, 0)),
            pl.BlockSpec((1, D, FD), lambda m: (m, 0, 0)),
            pl.BlockSpec((1, 1, FD), lambda m: (m, 0, 0)),
        ],
        out_specs=[
            pl.BlockSpec((1, SUB, B, 128), lambda m: (m, 0, 0, 0)),
            pl.BlockSpec((1, 1, SUB * B), lambda m: (m, 0, 0)),
        ],
        out_shape=[
            jax.ShapeDtypeStruct((3, SUB, B, 128), jnp.float32),
            jax.ShapeDtypeStruct((3, 1, SUB * B), jnp.int32),
        ],
    )(mi, Xs, Ws, bs)


def _tc_main(mi, FbT, projT, W1, b1, W2, b2):
    return pl.pallas_call(
        _main_kernel,
        grid=(3,),
        in_specs=[
            pl.BlockSpec((1, B), lambda m: (0, 0)),
            pl.BlockSpec((1, SUB, B, 128), lambda m: (m, 0, 0, 0)),
            pl.BlockSpec((1, SUB, B, 128), lambda m: (m, 0, 0, 0)),
            pl.BlockSpec((FD, FD), lambda m: (m, 0)),
            pl.BlockSpec((1, FD), lambda m: (0, 0)),
            pl.BlockSpec((FD, 1), lambda m: (0, 0)),
            pl.BlockSpec((1, 1), lambda m: (0, 0)),
        ],
        out_specs=pl.BlockSpec((B, 1), lambda m: (0, 0)),
        out_shape=jax.ShapeDtypeStruct((B, 1), jnp.float32),
        scratch_shapes=[pltpu.VMEM((B, FD), jnp.float32)],
    )(mi, FbT, projT, W1, b1, W2, b2)


def kernel(language, video, audio, missing_index, W_language, b_language,
           W_video, b_video, W_audio, b_audio, W1, b1, W2, b2):
    Xs = jnp.stack([language, video, audio])
    Ws = jnp.stack([W_language, W_video, W_audio])
    bs = jnp.stack([b_language, b_video, b_audio])[:, None, :]
    mi = missing_index.astype(jnp.int32).reshape(1, B)

    projT, src6 = _tc_proj_prep(mi, Xs, Ws, bs)
    idx = src6.reshape(1, 3 * B * SUB)
    FbT = _sc_gather(projT.reshape(3 * B * SUB, 128), idx)
    return _tc_main(mi, FbT.reshape(3, SUB, B, 128), projT, W1,
                    b1.reshape(1, FD), W2, b2.reshape(1, 1))
